# baseline (speedup 1.0000x reference)
import jax
import jax.numpy as jnp
from jax import lax
from jax.experimental import pallas as pl
from jax.experimental.pallas import tpu as pltpu
import jax.experimental.pallas.tpu_sc as plsc

B = 16384
D = 1792
EPS = 1e-5

SC_ROWS = 3072
TC_ROWS = B - SC_ROWS
BLK = 256

NW = 32
ROWS_W = SC_ROWS // NW
R = 16
CHW = R * D
NCH = ROWS_W // R
NV = D // 16
NACC = 4
UNROLL_A = 2
UNROLL_B = 4


def _sc_body(x_hbm, t_hbm, w_hbm, out_hbm,
             xb0, xb1, hb0, hb1, wb, tb,
             sin0, sin1, sout0, sout1):
    wid = lax.axis_index("s") * 2 + lax.axis_index("c")
    row0 = TC_ROWS + wid * ROWS_W
    base = row0 * D
    obase = wid * ROWS_W * D

    pltpu.sync_copy(w_hbm, wb)
    pltpu.sync_copy(t_hbm.at[pl.ds(row0, ROWS_W)], tb)

    zerov = jnp.full((16,), 0.0, jnp.float32)

    def start_in(c, xb, sem):
        pltpu.make_async_copy(
            x_hbm.at[pl.ds(base + c * CHW, CHW)], xb, sem).start()

    def wait_in(xb, sem):
        pltpu.make_async_copy(x_hbm.at[pl.ds(base, CHW)], xb, sem).wait()

    def start_out(c, hb, sem):
        pltpu.make_async_copy(
            hb, out_hbm.at[pl.ds(obase + c * CHW, CHW)], sem).start()

    def wait_out(hb, sem):
        pltpu.make_async_copy(hb, out_hbm.at[pl.ds(obase, CHW)], sem).wait()

    start_in(0, xb0, sin0)
    start_in(1, xb1, sin1)

    def do_chunk(c, xb, hb):
        tagv = tb[pl.ds(c * R, R)]
        inv_d = jnp.float32(1.0 / D)

        for r in range(R):
            bw = tagv[r] * D
            bx = r * D

            @plsc.parallel_loop(0, NV, step=NACC,
                                carry=((zerov,) * NACC, (zerov,) * NACC),
                                unroll=UNROLL_A)
            def bodyA(v, carry):
                s, ss = carry
                s, ss = list(s), list(ss)
                for u in range(NACC):
                    off = (v + u) * 16
                    xv = xb[pl.ds(bx + off, 16)]
                    wv = wb[pl.ds(bw + off, 16)]
                    h = xv + wv
                    hb[pl.ds(bx + off, 16)] = h
                    s[u] = s[u] + h
                    ss[u] = ss[u] + h * h
                return (tuple(s), tuple(ss))

            sl, ssl = bodyA
            sv = (sl[0] + sl[1]) + (sl[2] + sl[3])
            ssv = (ssl[0] + ssl[1]) + (ssl[2] + ssl[3])
            mean = jnp.sum(sv) * inv_d
            a = jnp.sum(ssv) * inv_d - mean * mean + EPS
            av = jnp.full((16,), a)
            i = plsc.bitcast(av, jnp.int32)
            i = jnp.int32(0x5F3759DF) - lax.shift_right_arithmetic(i, 1)
            y = plsc.bitcast(i, jnp.float32)
            half_a = av * 0.5
            for _ in range(4):
                y = y * (1.5 - half_a * y * y)
            rs = y
            mb = -(jnp.full((16,), mean) * rs)

            @plsc.parallel_loop(0, NV, carry=jnp.int32(0), unroll=UNROLL_B)
            def bodyB(v, carry):
                off = bx + v * 16
                h = hb[pl.ds(off, 16)]
                hb[pl.ds(off, 16)] = h * rs + mb
                return carry

            del bodyB

    def loop_k(k, _):
        wait_in(xb0, sin0)

        @pl.when(k > 0)
        def _():
            wait_out(hb0, sout0)

        do_chunk(2 * k, xb0, hb0)

        @pl.when(k < NCH // 2 - 1)
        def _():
            start_in(2 * k + 2, xb0, sin0)

        start_out(2 * k, hb0, sout0)

        wait_in(xb1, sin1)

        @pl.when(k > 0)
        def _():
            wait_out(hb1, sout1)

        do_chunk(2 * k + 1, xb1, hb1)

        @pl.when(k < NCH // 2 - 1)
        def _():
            start_in(2 * k + 3, xb1, sin1)

        start_out(2 * k + 1, hb1, sout1)
        return 0

    lax.fori_loop(0, NCH // 2, loop_k, 0)
    wait_out(hb0, sout0)
    wait_out(hb1, sout1)


def _tc_body(t_ref, w_ref, g_ref, b_ref, x_ref, o_ref):
    t = t_ref[...].astype(jnp.float32)
    w0 = w_ref[0:1, :]
    w1 = w_ref[1:2, :]
    q = w0 + t * (w1 - w0)
    h = x_ref[...] + q
    mean = jnp.mean(h, axis=1, keepdims=True)
    c = h - mean
    var = jnp.mean(c * c, axis=1, keepdims=True)
    o_ref[...] = c * lax.rsqrt(var + EPS) * g_ref[...] + b_ref[...]


def kernel(x, tags, W_emb, gamma, beta):
    tags = tags.astype(jnp.int32)

    mesh = plsc.VectorSubcoreMesh(core_axis_name="c", subcore_axis_name="s")
    sc = pl.kernel(
        _sc_body,
        out_type=jax.ShapeDtypeStruct((SC_ROWS * D,), jnp.float32),
        mesh=mesh,
        compiler_params=pltpu.CompilerParams(
            use_tc_tiling_on_sc=False, needs_layout_passes=False),
        scratch_types=[
            pltpu.VMEM((CHW,), jnp.float32),
            pltpu.VMEM((CHW,), jnp.float32),
            pltpu.VMEM((CHW,), jnp.float32),
            pltpu.VMEM((CHW,), jnp.float32),
            pltpu.VMEM((2 * D,), jnp.float32),
            pltpu.VMEM((ROWS_W,), jnp.int32),
            pltpu.SemaphoreType.DMA,
            pltpu.SemaphoreType.DMA,
            pltpu.SemaphoreType.DMA,
            pltpu.SemaphoreType.DMA,
        ],
    )
    sc_out = sc(x.reshape(B * D), tags, W_emb.reshape(2 * D))

    tcol = tags.reshape(B, 1)
    g2 = gamma.reshape(1, D)
    b2 = beta.reshape(1, D)
    tc_out = pl.pallas_call(
        _tc_body,
        grid=(TC_ROWS // BLK,),
        in_specs=[
            pl.BlockSpec((BLK, 1), lambda i: (i, 0)),
            pl.BlockSpec((2, D), lambda i: (0, 0)),
            pl.BlockSpec((1, D), lambda i: (0, 0)),
            pl.BlockSpec((1, D), lambda i: (0, 0)),
            pl.BlockSpec((BLK, D), lambda i: (i, 0)),
        ],
        out_specs=pl.BlockSpec((BLK, D), lambda i: (i, 0)),
        out_shape=jax.ShapeDtypeStruct((TC_ROWS, D), jnp.float32),
    )(tcol, W_emb, g2, b2, x)

    trivial = jnp.logical_and(jnp.all(gamma == 1.0), jnp.all(beta == 0.0))
    sc_fixed = lax.cond(
        trivial, lambda o: o,
        lambda o: (o.reshape(SC_ROWS, D) * gamma + beta).reshape(-1),
        sc_out)
    return jnp.concatenate([tc_out, sc_fixed.reshape(SC_ROWS, D)], axis=0)


# speedup vs baseline: 1.2269x; 1.2269x over previous
import jax
import jax.numpy as jnp
from jax import lax
from jax.experimental import pallas as pl
from jax.experimental.pallas import tpu as pltpu
import jax.experimental.pallas.tpu_sc as plsc

B = 16384
D = 1792
EPS = 1e-5

SC_ROWS = 3072
TC_ROWS = B - SC_ROWS
BLK = 256

NW = 32
ROWS_W = SC_ROWS // NW
R = 16
CHW = R * D
NCH = ROWS_W // R
NV = D // 16
NACC = 4
UNROLL_A = 2
UNROLL_B = 4


def _sc_body(x_hbm, t_hbm, w_hbm, out_hbm,
             xb0, xb1, hb0, hb1, wb, tb,
             sin0, sin1, sout0, sout1):
    wid = lax.axis_index("s") * 2 + lax.axis_index("c")
    row0 = TC_ROWS + wid * ROWS_W
    base = row0 * D
    obase = wid * ROWS_W * D

    pltpu.sync_copy(w_hbm, wb)
    pltpu.sync_copy(t_hbm.at[pl.ds(row0, ROWS_W)], tb)

    zerov = jnp.full((16,), 0.0, jnp.float32)

    def start_in(c, xb, sem):
        pltpu.make_async_copy(
            x_hbm.at[pl.ds(base + c * CHW, CHW)], xb, sem).start()

    def wait_in(xb, sem):
        pltpu.make_async_copy(x_hbm.at[pl.ds(base, CHW)], xb, sem).wait()

    def start_out(c, hb, sem):
        pltpu.make_async_copy(
            hb, out_hbm.at[pl.ds(obase + c * CHW, CHW)], sem).start()

    def wait_out(hb, sem):
        pltpu.make_async_copy(hb, out_hbm.at[pl.ds(obase, CHW)], sem).wait()

    start_in(0, xb0, sin0)
    start_in(1, xb1, sin1)

    def do_chunk(c, xb, hb):
        tagv = tb[pl.ds(c * R, R)]
        inv_d = jnp.float32(1.0 / D)

        for r in range(R):
            bw = tagv[r] * D
            bx = r * D

            @plsc.parallel_loop(0, NV, step=NACC,
                                carry=((zerov,) * NACC, (zerov,) * NACC),
                                unroll=UNROLL_A)
            def bodyA(v, carry):
                s, ss = carry
                s, ss = list(s), list(ss)
                for u in range(NACC):
                    off = (v + u) * 16
                    xv = xb[pl.ds(bx + off, 16)]
                    wv = wb[pl.ds(bw + off, 16)]
                    h = xv + wv
                    hb[pl.ds(bx + off, 16)] = h
                    s[u] = s[u] + h
                    ss[u] = ss[u] + h * h
                return (tuple(s), tuple(ss))

            sl, ssl = bodyA
            sv = (sl[0] + sl[1]) + (sl[2] + sl[3])
            ssv = (ssl[0] + ssl[1]) + (ssl[2] + ssl[3])
            mean = jnp.sum(sv) * inv_d
            a = jnp.sum(ssv) * inv_d - mean * mean + EPS
            av = jnp.full((16,), a)
            i = plsc.bitcast(av, jnp.int32)
            i = jnp.int32(0x5F3759DF) - lax.shift_right_arithmetic(i, 1)
            y = plsc.bitcast(i, jnp.float32)
            half_a = av * 0.5
            for _ in range(4):
                y = y * (1.5 - half_a * y * y)
            rs = y
            mb = -(jnp.full((16,), mean) * rs)

            @plsc.parallel_loop(0, NV, carry=jnp.int32(0), unroll=UNROLL_B)
            def bodyB(v, carry):
                off = bx + v * 16
                h = hb[pl.ds(off, 16)]
                hb[pl.ds(off, 16)] = h * rs + mb
                return carry

            del bodyB

    def loop_k(k, _):
        wait_in(xb0, sin0)

        @pl.when(k > 0)
        def _():
            wait_out(hb0, sout0)

        do_chunk(2 * k, xb0, hb0)

        @pl.when(k < NCH // 2 - 1)
        def _():
            start_in(2 * k + 2, xb0, sin0)

        start_out(2 * k, hb0, sout0)

        wait_in(xb1, sin1)

        @pl.when(k > 0)
        def _():
            wait_out(hb1, sout1)

        do_chunk(2 * k + 1, xb1, hb1)

        @pl.when(k < NCH // 2 - 1)
        def _():
            start_in(2 * k + 3, xb1, sin1)

        start_out(2 * k + 1, hb1, sout1)
        return 0

    lax.fori_loop(0, NCH // 2, loop_k, 0)
    wait_out(hb0, sout0)
    wait_out(hb1, sout1)


def _tc_body(t_ref, w_ref, g_ref, b_ref, x_ref, o_ref):
    t = t_ref[...].astype(jnp.float32)
    w0 = w_ref[0:1, :]
    w1 = w_ref[1:2, :]
    q = w0 + t * (w1 - w0)
    h = x_ref[...] + q
    mean = jnp.mean(h, axis=1, keepdims=True)
    c = h - mean
    var = jnp.mean(c * c, axis=1, keepdims=True)
    o_ref[...] = c * lax.rsqrt(var + EPS) * g_ref[...] + b_ref[...]


def kernel(x, tags, W_emb, gamma, beta):
    tags = tags.astype(jnp.int32)

    mesh = plsc.VectorSubcoreMesh(core_axis_name="c", subcore_axis_name="s")
    sc = pl.kernel(
        _sc_body,
        out_type=jax.ShapeDtypeStruct((SC_ROWS * D,), jnp.float32),
        mesh=mesh,
        compiler_params=pltpu.CompilerParams(
            use_tc_tiling_on_sc=False, needs_layout_passes=False),
        scratch_types=[
            pltpu.VMEM((CHW,), jnp.float32),
            pltpu.VMEM((CHW,), jnp.float32),
            pltpu.VMEM((CHW,), jnp.float32),
            pltpu.VMEM((CHW,), jnp.float32),
            pltpu.VMEM((2 * D,), jnp.float32),
            pltpu.VMEM((ROWS_W,), jnp.int32),
            pltpu.SemaphoreType.DMA,
            pltpu.SemaphoreType.DMA,
            pltpu.SemaphoreType.DMA,
            pltpu.SemaphoreType.DMA,
        ],
    )
    sc_out = sc(x.reshape(B * D), tags, W_emb.reshape(2 * D))

    tcol = tags.reshape(B, 1)
    g2 = gamma.reshape(1, D)
    b2 = beta.reshape(1, D)
    tc_full = pl.pallas_call(
        _tc_body,
        grid=(TC_ROWS // BLK,),
        in_specs=[
            pl.BlockSpec((BLK, 1), lambda i: (i, 0)),
            pl.BlockSpec((2, D), lambda i: (0, 0)),
            pl.BlockSpec((1, D), lambda i: (0, 0)),
            pl.BlockSpec((1, D), lambda i: (0, 0)),
            pl.BlockSpec((BLK, D), lambda i: (i, 0)),
        ],
        out_specs=pl.BlockSpec((BLK, D), lambda i: (i, 0)),
        out_shape=jax.ShapeDtypeStruct((B, D), jnp.float32),
    )(tcol, W_emb, g2, b2, x)

    trivial = jnp.logical_and(jnp.all(gamma == 1.0), jnp.all(beta == 0.0))
    sc_fixed = lax.cond(
        trivial, lambda o: o,
        lambda o: (o.reshape(SC_ROWS, D) * gamma + beta).reshape(-1),
        sc_out)
    return lax.dynamic_update_slice(
        tc_full, sc_fixed.reshape(SC_ROWS, D), (TC_ROWS, 0))


# speedup vs baseline: 2.4358x; 1.9852x over previous
import jax
import jax.numpy as jnp
from jax import lax
from jax.experimental import pallas as pl
from jax.experimental.pallas import tpu as pltpu
import jax.experimental.pallas.tpu_sc as plsc

B = 16384
D = 1792
EPS = 1e-5

SC_ROWS = 3072
TC_ROWS = B - SC_ROWS
BLK = 256

NW = 32
ROWS_W = SC_ROWS // NW
R = 16
NCH = ROWS_W // R
NV = D // 16
NACC = 4
UNROLL_A = 2
UNROLL_B = 4


def _sc_body(x_hbm, t_hbm, w_hbm, out_hbm,
             xb0, xb1, hb0, hb1, wb, tb,
             sin0, sin1, sout0, sout1):
    wid = lax.axis_index("s") * 2 + lax.axis_index("c")
    row0 = TC_ROWS + wid * ROWS_W
    orow0 = wid * ROWS_W

    pltpu.sync_copy(w_hbm, wb)
    pltpu.sync_copy(t_hbm.at[pl.ds(row0, ROWS_W)], tb)

    zerov = jnp.full((16,), 0.0, jnp.float32)

    def start_in(c, xb, sem):
        pltpu.make_async_copy(
            x_hbm.at[pl.ds(row0 + c * R, R), :], xb, sem).start()

    def wait_in(xb, sem):
        pltpu.make_async_copy(x_hbm.at[pl.ds(row0, R), :], xb, sem).wait()

    def start_out(c, hb, sem):
        pltpu.make_async_copy(
            hb, out_hbm.at[pl.ds(orow0 + c * R, R), :], sem).start()

    def wait_out(hb, sem):
        pltpu.make_async_copy(
            hb, out_hbm.at[pl.ds(orow0, R), :], sem).wait()

    start_in(0, xb0, sin0)
    start_in(1, xb1, sin1)

    def do_chunk(c, xb, hb):
        tagv = tb[pl.ds(c * R, R)]
        inv_d = jnp.float32(1.0 / D)

        for r in range(R):
            bw = tagv[r] * D

            @plsc.parallel_loop(0, NV, step=NACC,
                                carry=((zerov,) * NACC, (zerov,) * NACC),
                                unroll=UNROLL_A)
            def bodyA(v, carry):
                s, ss = carry
                s, ss = list(s), list(ss)
                for u in range(NACC):
                    off = (v + u) * 16
                    xv = xb[r, pl.ds(off, 16)]
                    wv = wb[pl.ds(bw + off, 16)]
                    h = xv + wv
                    hb[r, pl.ds(off, 16)] = h
                    s[u] = s[u] + h
                    ss[u] = ss[u] + h * h
                return (tuple(s), tuple(ss))

            sl, ssl = bodyA
            sv = (sl[0] + sl[1]) + (sl[2] + sl[3])
            ssv = (ssl[0] + ssl[1]) + (ssl[2] + ssl[3])
            mean = jnp.sum(sv) * inv_d
            a = jnp.sum(ssv) * inv_d - mean * mean + EPS
            av = jnp.full((16,), a)
            i = plsc.bitcast(av, jnp.int32)
            i = jnp.int32(0x5F3759DF) - lax.shift_right_arithmetic(i, 1)
            y = plsc.bitcast(i, jnp.float32)
            half_a = av * 0.5
            for _ in range(4):
                y = y * (1.5 - half_a * y * y)
            rs = y
            mb = -(jnp.full((16,), mean) * rs)

            @plsc.parallel_loop(0, NV, carry=jnp.int32(0), unroll=UNROLL_B)
            def bodyB(v, carry):
                off = v * 16
                h = hb[r, pl.ds(off, 16)]
                hb[r, pl.ds(off, 16)] = h * rs + mb
                return carry

            del bodyB

    def loop_k(k, _):
        wait_in(xb0, sin0)

        @pl.when(k > 0)
        def _():
            wait_out(hb0, sout0)

        do_chunk(2 * k, xb0, hb0)

        @pl.when(k < NCH // 2 - 1)
        def _():
            start_in(2 * k + 2, xb0, sin0)

        start_out(2 * k, hb0, sout0)

        wait_in(xb1, sin1)

        @pl.when(k > 0)
        def _():
            wait_out(hb1, sout1)

        do_chunk(2 * k + 1, xb1, hb1)

        @pl.when(k < NCH // 2 - 1)
        def _():
            start_in(2 * k + 3, xb1, sin1)

        start_out(2 * k + 1, hb1, sout1)
        return 0

    lax.fori_loop(0, NCH // 2, loop_k, 0)
    wait_out(hb0, sout0)
    wait_out(hb1, sout1)


def _patch_body(s_ref, g_ref, b_ref, dummy_ref, o_ref):
    del dummy_ref
    o_ref[...] = s_ref[...] * g_ref[...] + b_ref[...]


def _tc_body(t_ref, w_ref, g_ref, b_ref, x_ref, o_ref):
    t = t_ref[...].astype(jnp.float32)
    w0 = w_ref[0:1, :]
    w1 = w_ref[1:2, :]
    q = w0 + t * (w1 - w0)
    h = x_ref[...] + q
    mean = jnp.mean(h, axis=1, keepdims=True)
    c = h - mean
    var = jnp.mean(c * c, axis=1, keepdims=True)
    o_ref[...] = c * lax.rsqrt(var + EPS) * g_ref[...] + b_ref[...]


def kernel(x, tags, W_emb, gamma, beta):
    tags = tags.astype(jnp.int32)

    mesh = plsc.VectorSubcoreMesh(core_axis_name="c", subcore_axis_name="s")
    sc = pl.kernel(
        _sc_body,
        out_type=jax.ShapeDtypeStruct((SC_ROWS, D), jnp.float32),
        mesh=mesh,
        compiler_params=pltpu.CompilerParams(
            use_tc_tiling_on_sc=True, needs_layout_passes=False),
        scratch_types=[
            pltpu.VMEM((R, D), jnp.float32),
            pltpu.VMEM((R, D), jnp.float32),
            pltpu.VMEM((R, D), jnp.float32),
            pltpu.VMEM((R, D), jnp.float32),
            pltpu.VMEM((2 * D,), jnp.float32),
            pltpu.VMEM((ROWS_W,), jnp.int32),
            pltpu.SemaphoreType.DMA,
            pltpu.SemaphoreType.DMA,
            pltpu.SemaphoreType.DMA,
            pltpu.SemaphoreType.DMA,
        ],
    )
    sc_out = sc(x, tags, W_emb.reshape(2 * D))

    tcol = tags.reshape(B, 1)
    g2 = gamma.reshape(1, D)
    b2 = beta.reshape(1, D)
    tc_full = pl.pallas_call(
        _tc_body,
        grid=(TC_ROWS // BLK,),
        in_specs=[
            pl.BlockSpec((BLK, 1), lambda i: (i, 0)),
            pl.BlockSpec((2, D), lambda i: (0, 0)),
            pl.BlockSpec((1, D), lambda i: (0, 0)),
            pl.BlockSpec((1, D), lambda i: (0, 0)),
            pl.BlockSpec((BLK, D), lambda i: (i, 0)),
        ],
        out_specs=pl.BlockSpec((BLK, D), lambda i: (i, 0)),
        out_shape=jax.ShapeDtypeStruct((B, D), jnp.float32),
    )(tcol, W_emb, g2, b2, x)

    PBLK = 512
    return pl.pallas_call(
        _patch_body,
        grid=(SC_ROWS // PBLK,),
        in_specs=[
            pl.BlockSpec((PBLK, D), lambda i: (i, 0)),
            pl.BlockSpec((1, D), lambda i: (0, 0)),
            pl.BlockSpec((1, D), lambda i: (0, 0)),
            pl.BlockSpec((8, 128), lambda i: (0, 0)),
        ],
        out_specs=pl.BlockSpec(
            (PBLK, D), lambda i: (TC_ROWS // PBLK + i, 0)),
        out_shape=jax.ShapeDtypeStruct((B, D), jnp.float32),
        input_output_aliases={3: 0},
    )(sc_out, g2, b2, tc_full)


# speedup vs baseline: 2.4516x; 1.0065x over previous
import jax
import jax.numpy as jnp
from jax import lax
from jax.experimental import pallas as pl
from jax.experimental.pallas import tpu as pltpu
import jax.experimental.pallas.tpu_sc as plsc

B = 16384
D = 1792
EPS = 1e-5

SC_ROWS = 2048
TC_ROWS = B - SC_ROWS
BLK = 256

NW = 32
ROWS_W = SC_ROWS // NW
R = 16
NCH = ROWS_W // R
NV = D // 16
NACC = 4
UNROLL_A = 2
UNROLL_B = 4


def _sc_body(x_hbm, t_hbm, w_hbm, out_hbm,
             xb0, xb1, hb0, hb1, wb, tb,
             sin0, sin1, sout0, sout1):
    wid = lax.axis_index("s") * 2 + lax.axis_index("c")
    row0 = TC_ROWS + wid * ROWS_W
    orow0 = wid * ROWS_W

    pltpu.sync_copy(w_hbm, wb)
    pltpu.sync_copy(t_hbm.at[pl.ds(row0, ROWS_W)], tb)

    zerov = jnp.full((16,), 0.0, jnp.float32)

    def start_in(c, xb, sem):
        pltpu.make_async_copy(
            x_hbm.at[pl.ds(row0 + c * R, R), :], xb, sem).start()

    def wait_in(xb, sem):
        pltpu.make_async_copy(x_hbm.at[pl.ds(row0, R), :], xb, sem).wait()

    def start_out(c, hb, sem):
        pltpu.make_async_copy(
            hb, out_hbm.at[pl.ds(orow0 + c * R, R), :], sem).start()

    def wait_out(hb, sem):
        pltpu.make_async_copy(
            hb, out_hbm.at[pl.ds(orow0, R), :], sem).wait()

    start_in(0, xb0, sin0)
    start_in(1, xb1, sin1)

    def do_chunk(c, xb, hb):
        tagv = tb[pl.ds(c * R, R)]
        inv_d = jnp.float32(1.0 / D)

        for r in range(R):
            bw = tagv[r] * D

            @plsc.parallel_loop(0, NV, step=NACC,
                                carry=((zerov,) * NACC, (zerov,) * NACC),
                                unroll=UNROLL_A)
            def bodyA(v, carry):
                s, ss = carry
                s, ss = list(s), list(ss)
                for u in range(NACC):
                    off = (v + u) * 16
                    xv = xb[r, pl.ds(off, 16)]
                    wv = wb[pl.ds(bw + off, 16)]
                    h = xv + wv
                    hb[r, pl.ds(off, 16)] = h
                    s[u] = s[u] + h
                    ss[u] = ss[u] + h * h
                return (tuple(s), tuple(ss))

            sl, ssl = bodyA
            sv = (sl[0] + sl[1]) + (sl[2] + sl[3])
            ssv = (ssl[0] + ssl[1]) + (ssl[2] + ssl[3])
            mean = jnp.sum(sv) * inv_d
            a = jnp.sum(ssv) * inv_d - mean * mean + EPS
            av = jnp.full((16,), a)
            i = plsc.bitcast(av, jnp.int32)
            i = jnp.int32(0x5F3759DF) - lax.shift_right_arithmetic(i, 1)
            y = plsc.bitcast(i, jnp.float32)
            half_a = av * 0.5
            for _ in range(4):
                y = y * (1.5 - half_a * y * y)
            rs = y
            mb = -(jnp.full((16,), mean) * rs)

            @plsc.parallel_loop(0, NV, carry=jnp.int32(0), unroll=UNROLL_B)
            def bodyB(v, carry):
                off = v * 16
                h = hb[r, pl.ds(off, 16)]
                hb[r, pl.ds(off, 16)] = h * rs + mb
                return carry

            del bodyB

    def loop_k(k, _):
        wait_in(xb0, sin0)

        @pl.when(k > 0)
        def _():
            wait_out(hb0, sout0)

        do_chunk(2 * k, xb0, hb0)

        @pl.when(k < NCH // 2 - 1)
        def _():
            start_in(2 * k + 2, xb0, sin0)

        start_out(2 * k, hb0, sout0)

        wait_in(xb1, sin1)

        @pl.when(k > 0)
        def _():
            wait_out(hb1, sout1)

        do_chunk(2 * k + 1, xb1, hb1)

        @pl.when(k < NCH // 2 - 1)
        def _():
            start_in(2 * k + 3, xb1, sin1)

        start_out(2 * k + 1, hb1, sout1)
        return 0

    lax.fori_loop(0, NCH // 2, loop_k, 0)
    wait_out(hb0, sout0)
    wait_out(hb1, sout1)


def _patch_body(s_ref, g_ref, b_ref, dummy_ref, o_ref):
    del dummy_ref
    o_ref[...] = s_ref[...] * g_ref[...] + b_ref[...]


def _tc_body(t_ref, w_ref, g_ref, b_ref, x_ref, o_ref):
    t = t_ref[...].astype(jnp.float32)
    w0 = w_ref[0:1, :]
    w1 = w_ref[1:2, :]
    q = w0 + t * (w1 - w0)
    h = x_ref[...] + q
    mean = jnp.mean(h, axis=1, keepdims=True)
    c = h - mean
    var = jnp.mean(c * c, axis=1, keepdims=True)
    o_ref[...] = c * lax.rsqrt(var + EPS) * g_ref[...] + b_ref[...]


def kernel(x, tags, W_emb, gamma, beta):
    tags = tags.astype(jnp.int32)

    mesh = plsc.VectorSubcoreMesh(core_axis_name="c", subcore_axis_name="s")
    sc = pl.kernel(
        _sc_body,
        out_type=jax.ShapeDtypeStruct((SC_ROWS, D), jnp.float32),
        mesh=mesh,
        compiler_params=pltpu.CompilerParams(
            use_tc_tiling_on_sc=True, needs_layout_passes=False),
        scratch_types=[
            pltpu.VMEM((R, D), jnp.float32),
            pltpu.VMEM((R, D), jnp.float32),
            pltpu.VMEM((R, D), jnp.float32),
            pltpu.VMEM((R, D), jnp.float32),
            pltpu.VMEM((2 * D,), jnp.float32),
            pltpu.VMEM((ROWS_W,), jnp.int32),
            pltpu.SemaphoreType.DMA,
            pltpu.SemaphoreType.DMA,
            pltpu.SemaphoreType.DMA,
            pltpu.SemaphoreType.DMA,
        ],
    )
    sc_out = sc(x, tags, W_emb.reshape(2 * D))

    tcol = tags.reshape(B, 1)
    g2 = gamma.reshape(1, D)
    b2 = beta.reshape(1, D)
    tc_full = pl.pallas_call(
        _tc_body,
        grid=(TC_ROWS // BLK,),
        in_specs=[
            pl.BlockSpec((BLK, 1), lambda i: (i, 0)),
            pl.BlockSpec((2, D), lambda i: (0, 0)),
            pl.BlockSpec((1, D), lambda i: (0, 0)),
            pl.BlockSpec((1, D), lambda i: (0, 0)),
            pl.BlockSpec((BLK, D), lambda i: (i, 0)),
        ],
        out_specs=pl.BlockSpec((BLK, D), lambda i: (i, 0)),
        out_shape=jax.ShapeDtypeStruct((B, D), jnp.float32),
    )(tcol, W_emb, g2, b2, x)

    PBLK = 512
    return pl.pallas_call(
        _patch_body,
        grid=(SC_ROWS // PBLK,),
        in_specs=[
            pl.BlockSpec((PBLK, D), lambda i: (i, 0)),
            pl.BlockSpec((1, D), lambda i: (0, 0)),
            pl.BlockSpec((1, D), lambda i: (0, 0)),
            pl.BlockSpec((8, 128), lambda i: (0, 0)),
        ],
        out_specs=pl.BlockSpec(
            (PBLK, D), lambda i: (TC_ROWS // PBLK + i, 0)),
        out_shape=jax.ShapeDtypeStruct((B, D), jnp.float32),
        input_output_aliases={3: 0},
    )(sc_out, g2, b2, tc_full)


# speedup vs baseline: 2.5205x; 1.0281x over previous
import jax
import jax.numpy as jnp
from jax import lax
from jax.experimental import pallas as pl
from jax.experimental.pallas import tpu as pltpu
import jax.experimental.pallas.tpu_sc as plsc

B = 16384
D = 1792
EPS = 1e-5

SC_ROWS = 5120
TC_ROWS = B - SC_ROWS
BLK = 512

NW = 32
ROWS_W = SC_ROWS // NW
R = 16
NCH = ROWS_W // R
NV = D // 16
NACC = 4
UNROLL_A = 2
UNROLL_B = 4


def _sc_body(x_hbm, t_hbm, w_hbm, out_hbm,
             xb0, xb1, hb0, hb1, wb, tb,
             sin0, sin1, sout0, sout1):
    wid = lax.axis_index("s") * 2 + lax.axis_index("c")
    row0 = TC_ROWS + wid * ROWS_W
    orow0 = wid * ROWS_W

    pltpu.sync_copy(w_hbm, wb)
    pltpu.sync_copy(t_hbm.at[pl.ds(row0, ROWS_W)], tb)

    zerov = jnp.full((16,), 0.0, jnp.float32)

    def start_in(c, xb, sem):
        pltpu.make_async_copy(
            x_hbm.at[pl.ds(row0 + c * R, R), :], xb, sem).start()

    def wait_in(xb, sem):
        pltpu.make_async_copy(x_hbm.at[pl.ds(row0, R), :], xb, sem).wait()

    def start_out(c, hb, sem):
        pltpu.make_async_copy(
            hb, out_hbm.at[pl.ds(orow0 + c * R, R), :], sem).start()

    def wait_out(hb, sem):
        pltpu.make_async_copy(
            hb, out_hbm.at[pl.ds(orow0, R), :], sem).wait()

    start_in(0, xb0, sin0)
    start_in(1, xb1, sin1)

    def do_chunk(c, xb, hb):
        tagv = tb[pl.ds(c * R, R)]
        inv_d = jnp.float32(1.0 / D)

        for r in range(R):
            bw = tagv[r] * D

            @plsc.parallel_loop(0, NV, step=NACC,
                                carry=((zerov,) * NACC, (zerov,) * NACC),
                                unroll=UNROLL_A)
            def bodyA(v, carry):
                s, ss = carry
                s, ss = list(s), list(ss)
                for u in range(NACC):
                    off = (v + u) * 16
                    xv = xb[r, pl.ds(off, 16)]
                    wv = wb[pl.ds(bw + off, 16)]
                    h = xv + wv
                    hb[r, pl.ds(off, 16)] = h
                    s[u] = s[u] + h
                    ss[u] = ss[u] + h * h
                return (tuple(s), tuple(ss))

            sl, ssl = bodyA
            sv = (sl[0] + sl[1]) + (sl[2] + sl[3])
            ssv = (ssl[0] + ssl[1]) + (ssl[2] + ssl[3])
            mean = jnp.sum(sv) * inv_d
            a = jnp.sum(ssv) * inv_d - mean * mean + EPS
            av = jnp.full((16,), a)
            i = plsc.bitcast(av, jnp.int32)
            i = jnp.int32(0x5F3759DF) - lax.shift_right_arithmetic(i, 1)
            y = plsc.bitcast(i, jnp.float32)
            half_a = av * 0.5
            for _ in range(4):
                y = y * (1.5 - half_a * y * y)
            rs = y
            mb = -(jnp.full((16,), mean) * rs)

            @plsc.parallel_loop(0, NV, carry=jnp.int32(0), unroll=UNROLL_B)
            def bodyB(v, carry):
                off = v * 16
                h = hb[r, pl.ds(off, 16)]
                hb[r, pl.ds(off, 16)] = h * rs + mb
                return carry

            del bodyB

    def loop_k(k, _):
        wait_in(xb0, sin0)

        @pl.when(k > 0)
        def _():
            wait_out(hb0, sout0)

        do_chunk(2 * k, xb0, hb0)

        @pl.when(k < NCH // 2 - 1)
        def _():
            start_in(2 * k + 2, xb0, sin0)

        start_out(2 * k, hb0, sout0)

        wait_in(xb1, sin1)

        @pl.when(k > 0)
        def _():
            wait_out(hb1, sout1)

        do_chunk(2 * k + 1, xb1, hb1)

        @pl.when(k < NCH // 2 - 1)
        def _():
            start_in(2 * k + 3, xb1, sin1)

        start_out(2 * k + 1, hb1, sout1)
        return 0

    lax.fori_loop(0, NCH // 2, loop_k, 0)
    wait_out(hb0, sout0)
    wait_out(hb1, sout1)


def _patch_body(s_ref, g_ref, b_ref, dummy_ref, o_ref):
    del dummy_ref
    o_ref[...] = s_ref[...] * g_ref[...] + b_ref[...]


def _tc_body(t_ref, w_ref, g_ref, b_ref, x_ref, o_ref):
    t = t_ref[...].astype(jnp.float32)
    w0 = w_ref[0:1, :]
    w1 = w_ref[1:2, :]
    q = w0 + t * (w1 - w0)
    h = x_ref[...] + q
    mean = jnp.mean(h, axis=1, keepdims=True)
    c = h - mean
    var = jnp.mean(c * c, axis=1, keepdims=True)
    o_ref[...] = c * lax.rsqrt(var + EPS) * g_ref[...] + b_ref[...]


def kernel(x, tags, W_emb, gamma, beta):
    tags = tags.astype(jnp.int32)

    mesh = plsc.VectorSubcoreMesh(core_axis_name="c", subcore_axis_name="s")
    sc = pl.kernel(
        _sc_body,
        out_type=jax.ShapeDtypeStruct((SC_ROWS, D), jnp.float32),
        mesh=mesh,
        compiler_params=pltpu.CompilerParams(
            use_tc_tiling_on_sc=True, needs_layout_passes=False),
        scratch_types=[
            pltpu.VMEM((R, D), jnp.float32),
            pltpu.VMEM((R, D), jnp.float32),
            pltpu.VMEM((R, D), jnp.float32),
            pltpu.VMEM((R, D), jnp.float32),
            pltpu.VMEM((2 * D,), jnp.float32),
            pltpu.VMEM((ROWS_W,), jnp.int32),
            pltpu.SemaphoreType.DMA,
            pltpu.SemaphoreType.DMA,
            pltpu.SemaphoreType.DMA,
            pltpu.SemaphoreType.DMA,
        ],
    )
    sc_out = sc(x, tags, W_emb.reshape(2 * D))

    tcol = tags.reshape(B, 1)
    g2 = gamma.reshape(1, D)
    b2 = beta.reshape(1, D)
    tc_full = pl.pallas_call(
        _tc_body,
        grid=(TC_ROWS // BLK,),
        in_specs=[
            pl.BlockSpec((BLK, 1), lambda i: (i, 0)),
            pl.BlockSpec((2, D), lambda i: (0, 0)),
            pl.BlockSpec((1, D), lambda i: (0, 0)),
            pl.BlockSpec((1, D), lambda i: (0, 0)),
            pl.BlockSpec((BLK, D), lambda i: (i, 0)),
        ],
        out_specs=pl.BlockSpec((BLK, D), lambda i: (i, 0)),
        out_shape=jax.ShapeDtypeStruct((B, D), jnp.float32),
    )(tcol, W_emb, g2, b2, x)

    PBLK = 512
    return pl.pallas_call(
        _patch_body,
        grid=(SC_ROWS // PBLK,),
        in_specs=[
            pl.BlockSpec((PBLK, D), lambda i: (i, 0)),
            pl.BlockSpec((1, D), lambda i: (0, 0)),
            pl.BlockSpec((1, D), lambda i: (0, 0)),
            pl.BlockSpec((8, 128), lambda i: (0, 0)),
        ],
        out_specs=pl.BlockSpec(
            (PBLK, D), lambda i: (TC_ROWS // PBLK + i, 0)),
        out_shape=jax.ShapeDtypeStruct((B, D), jnp.float32),
        input_output_aliases={3: 0},
    )(sc_out, g2, b2, tc_full)
